# baseline (speedup 1.0000x reference)
import functools

import jax
import jax.numpy as jnp
from jax import lax
from jax.experimental import pallas as pl
from jax.experimental.pallas import tpu as pltpu
from jax.experimental.pallas import tpu_sc as plsc

N = 10000
E = 160000
NET = 2
DIN = 128
EMB = 256

NSC = 2
NTILE = 16
B = 96
NCHUNK = 54
E_PAD = NSC * NTILE * NCHUNK * B
ACC = 10240
RPT = ACC // NTILE
ZR = 160
NPASS = 2 * NET
F1Q = 136

BN = 400
NBLK = N // BN


def _sc_hist(hidx, zrow, ones_h):
    mesh = plsc.VectorSubcoreMesh(core_axis_name="c", subcore_axis_name="s",
                                  num_cores=NSC, num_subcores=NTILE)
    PT = NPASS * ACC // NTILE

    @functools.partial(
        pl.kernel,
        out_type=jax.ShapeDtypeStruct((NSC * NPASS * ACC,), jnp.float32),
        mesh=mesh,
        scratch_types=[
            pltpu.VMEM_SHARED((NPASS * ACC,), jnp.float32),
            pltpu.VMEM((NCHUNK, B), jnp.int32),
            pltpu.VMEM((B,), jnp.float32),
        ],
    )
    def k(hidx_h, zrow_h, ones_hbm, out, acc, sv, ones_v):
        cid = lax.axis_index("c")
        sid = lax.axis_index("s")
        r0 = sid * RPT
        pltpu.sync_copy(ones_hbm, ones_v)
        pltpu.sync_copy(zrow_h, acc.at[pl.ds(sid * PT, PT)])
        plsc.subcore_barrier()
        for p in range(NPASS):
            pltpu.sync_copy(hidx_h.at[p, cid, sid], sv)

            def body(j, carry):
                pltpu.sync_copy(ones_v, acc.at[sv.at[j]], add=True)
                return carry

            lax.fori_loop(0, NCHUNK, body, 0)
        plsc.subcore_barrier()
        for p in range(NPASS):
            pltpu.sync_copy(
                acc.at[pl.ds(p * ACC + r0, RPT)],
                out.at[pl.ds((cid * NPASS + p) * ACC + r0, RPT)])

    return k(hidx, zrow, ones_h)


def _sc_spmm(tab, gidx, sidx, zrows, F):
    mesh = plsc.VectorSubcoreMesh(core_axis_name="c", subcore_axis_name="s",
                                  num_cores=NSC, num_subcores=NTILE)

    @functools.partial(
        pl.kernel,
        out_type=jax.ShapeDtypeStruct((NSC, NPASS, ACC, F), jnp.float32),
        mesh=mesh,
        scratch_types=[
            pltpu.VMEM_SHARED((ACC, F), jnp.float32),
            pltpu.VMEM((NCHUNK, B), jnp.int32),
            pltpu.VMEM((NCHUNK, B), jnp.int32),
            pltpu.VMEM((B, F), jnp.float32),
            pltpu.VMEM((B, F), jnp.float32),
            pltpu.SemaphoreType.DMA,
            pltpu.SemaphoreType.DMA,
        ],
        compiler_params=pltpu.CompilerParams(use_tc_tiling_on_sc=False),
    )
    def k(tab_h, gidx_h, sidx_h, zrows_h, out, acc, gv, sv, rows0, rows1,
          gsem, ssem):
        cid = lax.axis_index("c")
        sid = lax.axis_index("s")
        r0 = sid * RPT
        rows = (rows0, rows1)
        for p in range(NPASS):
            pltpu.sync_copy(gidx_h.at[p, cid, sid], gv)
            pltpu.sync_copy(sidx_h.at[p, cid, sid], sv)
            for z in range(RPT // ZR):
                pltpu.sync_copy(zrows_h, acc.at[pl.ds(r0 + z * ZR, ZR)])
            plsc.subcore_barrier()

            pltpu.async_copy(tab_h.at[gv.at[0]], rows0, gsem).wait()

            def body(k2, carry):
                for t in range(2):
                    j = 2 * k2 + 1 + t
                    buf = rows[(1 + t) % 2]
                    prev = rows[t % 2]
                    pltpu.async_copy(tab_h.at[gv.at[j]], buf, gsem)
                    pltpu.async_copy(prev, acc.at[sv.at[j - 1]], ssem,
                                     add=True)
                    pltpu.make_async_copy(tab_h.at[gv.at[j]], buf,
                                          gsem).wait()
                    pltpu.make_async_copy(prev, acc.at[sv.at[j - 1]],
                                          ssem).wait()
                return carry

            lax.fori_loop(0, (NCHUNK - 2) // 2, body, 0)
            jl = NCHUNK - 1
            pltpu.async_copy(tab_h.at[gv.at[jl]], rows1, gsem)
            pltpu.async_copy(rows0, acc.at[sv.at[jl - 1]], ssem, add=True)
            pltpu.make_async_copy(tab_h.at[gv.at[jl]], rows1, gsem).wait()
            pltpu.make_async_copy(rows0, acc.at[sv.at[jl - 1]], ssem).wait()
            pltpu.sync_copy(rows1, acc.at[sv.at[jl]], add=True)
            plsc.subcore_barrier()
            pltpu.sync_copy(acc.at[pl.ds(r0, RPT)],
                            out.at[cid, p, pl.ds(r0, RPT)])

    return k(tab, gidx, sidx, zrows)


def _tc_degpost(degPT):
    HB = 1280

    def f(dp_ref, meta_ref):
        dp = dp_ref[...]
        d = dp[:, 0, :] + dp[:, 1, :]
        invd = jnp.where(d > 0.0,
                         1.0 / jnp.sqrt(jnp.maximum(d, 1.0)), 0.0)
        degc = jnp.minimum(d, float(EMB - 1))
        meta_ref[...] = jnp.concatenate([invd, degc], axis=1)

    return pl.pallas_call(
        f,
        grid=(ACC // HB,),
        in_specs=[pl.BlockSpec((HB, NSC, NPASS), lambda j: (j, 0, 0))],
        out_specs=pl.BlockSpec((HB, 8), lambda j: (j, 0)),
        out_shape=jax.ShapeDtypeStruct((ACC, 8), jnp.float32),
    )(degPT)


def _tc_pre_a(x, meta, wproj):
    din = x.shape[1]

    def f(x_ref, m_ref, wp_ref, t1q_ref):
        xb = x_ref[...]
        m = m_ref[...]
        s = 1.0 / (jnp.sum(jnp.abs(xb), axis=1, keepdims=True) + 1e-12)
        x1 = xb * s
        xx = xb * xb
        q2 = jnp.dot(xx, wp_ref[...], preferred_element_type=jnp.float32)
        qo = q2[:, 0:1] * s * s
        qi = q2[:, 1:2] * s * s
        for i in range(NET):
            a = m[:, 2 * i:2 * i + 1]
            b = m[:, 2 * i + 1:2 * i + 2]
            t1q_ref[2 * i] = jnp.concatenate(
                [b * x1, jnp.broadcast_to(b * qo, (BN, F1Q - din))], axis=1)
            t1q_ref[2 * i + 1] = jnp.concatenate(
                [a * x1, jnp.broadcast_to(a * qi, (BN, F1Q - din))], axis=1)

    return pl.pallas_call(
        f,
        grid=(NBLK,),
        in_specs=[
            pl.BlockSpec((BN, din), lambda j: (j, 0)),
            pl.BlockSpec((BN, 8), lambda j: (j, 0)),
            pl.BlockSpec((din, 128), lambda j: (0, 0)),
        ],
        out_specs=pl.BlockSpec((NPASS, BN, F1Q), lambda j: (0, j, 0)),
        out_shape=jax.ShapeDtypeStruct((NPASS, N, F1Q), jnp.float32),
    )(x, meta, wproj)


def _tc_pre_b(x, meta, Ws2d, Wd2s, dout):
    din = x.shape[1]

    def f(x_ref, m_ref, ws_ref, wd_ref, txw_ref):
        xb = x_ref[...]
        m = m_ref[...]
        xws = jnp.dot(xb, ws_ref[...], preferred_element_type=jnp.float32)
        xwd = jnp.dot(xb, wd_ref[...], preferred_element_type=jnp.float32)
        for i in range(NET):
            a = m[:, 2 * i:2 * i + 1]
            b = m[:, 2 * i + 1:2 * i + 2]
            txw_ref[2 * i] = b * xws
            txw_ref[2 * i + 1] = a * xwd

    return pl.pallas_call(
        f,
        grid=(NBLK,),
        in_specs=[
            pl.BlockSpec((BN, din), lambda j: (j, 0)),
            pl.BlockSpec((BN, 8), lambda j: (j, 0)),
            pl.BlockSpec((din, dout), lambda j: (0, 0)),
            pl.BlockSpec((din, dout), lambda j: (0, 0)),
        ],
        out_specs=pl.BlockSpec((NPASS, BN, dout), lambda j: (0, j, 0)),
        out_shape=jax.ShapeDtypeStruct((NPASS, N, dout), jnp.float32),
    )(x, meta, Ws2d, Wd2s)


def _tc_gates(x, meta, mpack, P1q, out_emb, in_emb, wproj, wprojT, bpack):
    din = x.shape[1]

    def f(x_ref, m_ref, mp_ref, p1_ref, oe_ref, ie_ref, wp_ref, wt_ref,
          bp_ref, g_ref):
        xb = x_ref[...]
        m = m_ref[...]
        mp = mp_ref[...]
        bp = bp_ref[...]
        jblk = pl.program_id(0)
        s = 1.0 / (jnp.sum(jnp.abs(xb), axis=1, keepdims=True) + 1e-12)
        x1 = xb * s
        xx = xb * xb
        q2 = jnp.dot(xx, wp_ref[...], preferred_element_type=jnp.float32)
        qo = q2[:, 0:1] * s * s
        qi = q2[:, 1:2] * s * s
        rowi = lax.broadcasted_iota(jnp.int32, (BN, 1), 0) + jblk * BN
        tmd = 1.0 + (rowi == N - 1).astype(jnp.float32)
        embp_o = jnp.dot(oe_ref[...], wp_ref[...],
                         preferred_element_type=jnp.float32)[:, 0:1]
        embp_i = jnp.dot(ie_ref[...], wp_ref[...],
                         preferred_element_type=jnp.float32)[:, 1:2]
        eff_tau = jnp.exp(bp[3, 2]) + 0.1
        emb_iota = lax.broadcasted_iota(jnp.int32, (BN, EMB), 1)
        cols = []
        for i in range(NET):
            a = m[:, 2 * i:2 * i + 1]
            b = m[:, 2 * i + 1:2 * i + 2]
            G1A = p1_ref[0, 2 * i] + p1_ref[1, 2 * i]
            G1B = p1_ref[0, 2 * i + 1] + p1_ref[1, 2 * i + 1]
            eout = (tmd * qo + a * G1A[:, din:din + 1]
                    - 2.0 * jnp.sum(a * G1A[:, :din] * x1 * wt_ref[0:1, :],
                                    axis=1, keepdims=True))
            ein = (tmd * qi + b * G1B[:, din:din + 1]
                   - 2.0 * jnp.sum(b * G1B[:, :din] * x1 * wt_ref[1:2, :],
                                   axis=1, keepdims=True))
            oh_o = (m[:, 4 + 2 * i:5 + 2 * i].astype(jnp.int32)
                    == emb_iota).astype(jnp.float32)
            oh_i = (m[:, 5 + 2 * i:6 + 2 * i].astype(jnp.int32)
                    == emb_iota).astype(jnp.float32)
            enc_o = jnp.dot(oh_o, embp_o, preferred_element_type=jnp.float32)
            enc_i = jnp.dot(oh_i, embp_i, preferred_element_type=jnp.float32)
            co = -eout + enc_o + bp[3, 0]
            ci = -ein + enc_i + bp[3, 1]
            u = co / eff_tau
            v = ci / eff_tau
            mx = jnp.maximum(u, v)
            eu = jnp.exp(u - mx)
            ev = jnp.exp(v - mx)
            z = eu + ev
            cols.append((eu / z) * mp[:, 4 * i:4 * i + 1]
                        + mp[:, 4 * i + 1:4 * i + 2])
            cols.append((ev / z) * mp[:, 4 * i + 2:4 * i + 3]
                        + mp[:, 4 * i + 3:4 * i + 4])
        cols.append(jnp.zeros((BN, 8 - 2 * NET), jnp.float32))
        g_ref[...] = jnp.concatenate(cols, axis=1)

    return pl.pallas_call(
        f,
        grid=(NBLK,),
        in_specs=[
            pl.BlockSpec((BN, din), lambda j: (j, 0)),
            pl.BlockSpec((BN, 8), lambda j: (j, 0)),
            pl.BlockSpec((BN, 8), lambda j: (j, 0)),
            pl.BlockSpec((NSC, NPASS, BN, F1Q), lambda j: (0, 0, j, 0)),
            pl.BlockSpec((EMB, din), lambda j: (0, 0)),
            pl.BlockSpec((EMB, din), lambda j: (0, 0)),
            pl.BlockSpec((din, 128), lambda j: (0, 0)),
            pl.BlockSpec((8, din), lambda j: (0, 0)),
            pl.BlockSpec((8, 128), lambda j: (0, 0)),
        ],
        out_specs=pl.BlockSpec((BN, 8), lambda j: (j, 0)),
        out_shape=jax.ShapeDtypeStruct((N, 8), jnp.float32),
    )(x, meta, mpack, P1q, out_emb, in_emb, wproj, wprojT, bpack)


def _tc_combine(x, meta, gates, Pxw, Wfc, bpack, dout):
    din = x.shape[1]

    def f(x_ref, m_ref, g_ref, pw_ref, wfc_ref, bp_ref, out_ref, st_ref):
        xb = x_ref[...]
        m = m_ref[...]
        g = g_ref[...]
        bp = bp_ref[...]
        alpha = bp[3, 3]
        acc = alpha * (jnp.dot(xb, wfc_ref[...],
                               preferred_element_type=jnp.float32)
                       + bp[2:3, :dout])
        for i in range(NET):
            a = m[:, 2 * i:2 * i + 1]
            b = m[:, 2 * i + 1:2 * i + 2]
            GWA = pw_ref[0, 2 * i] + pw_ref[1, 2 * i]
            GWB = pw_ref[0, 2 * i + 1] + pw_ref[1, 2 * i + 1]
            out_nei = a * GWA + bp[0:1, :dout]
            in_nei = b * GWB + bp[1:2, :dout]
            acc = acc + (1.0 / NET) * (g[:, 2 * i:2 * i + 1] * out_nei
                                       + g[:, 2 * i + 1:2 * i + 2] * in_nei)
        out_ref[...] = acc
        st_ref[...] = jnp.zeros((1, 8, dout), jnp.float32)
        st_ref[0, 0] = jnp.sum(acc, axis=0)
        st_ref[0, 1] = jnp.sum(acc * acc, axis=0)

    return pl.pallas_call(
        f,
        grid=(NBLK,),
        in_specs=[
            pl.BlockSpec((BN, din), lambda j: (j, 0)),
            pl.BlockSpec((BN, 8), lambda j: (j, 0)),
            pl.BlockSpec((BN, 8), lambda j: (j, 0)),
            pl.BlockSpec((NSC, NPASS, BN, dout), lambda j: (0, 0, j, 0)),
            pl.BlockSpec((din, dout), lambda j: (0, 0)),
            pl.BlockSpec((8, 128), lambda j: (0, 0)),
        ],
        out_specs=[
            pl.BlockSpec((BN, dout), lambda j: (j, 0)),
            pl.BlockSpec((1, 8, dout), lambda j: (j, 0, 0)),
        ],
        out_shape=[
            jax.ShapeDtypeStruct((N, dout), jnp.float32),
            jax.ShapeDtypeStruct((NBLK, 8, dout), jnp.float32),
        ],
    )(x, meta, gates, Pxw, Wfc, bpack)


def _tc_post_c(out_raw, st, bnpack, dout, relu):
    def f(o_ref, st_ref, bn_ref, y_ref):
        stats = st_ref[...]
        mean = jnp.sum(stats[:, 0, :], axis=0, keepdims=True) / N
        msq = jnp.sum(stats[:, 1, :], axis=0, keepdims=True) / N
        var = msq - mean * mean
        inv = 1.0 / jnp.sqrt(var + 1e-5)
        bn = bn_ref[...]
        y = (o_ref[...] - mean) * inv * bn[0:1, :dout] + bn[1:2, :dout]
        if relu:
            y = jnp.maximum(y, 0.0)
        y_ref[...] = y

    return pl.pallas_call(
        f,
        grid=(NBLK,),
        in_specs=[
            pl.BlockSpec((BN, dout), lambda j: (j, 0)),
            pl.BlockSpec((NBLK, 8, dout), lambda j: (0, 0, 0)),
            pl.BlockSpec((8, 128), lambda j: (0, 0)),
        ],
        out_specs=pl.BlockSpec((BN, dout), lambda j: (j, 0)),
        out_shape=jax.ShapeDtypeStruct((N, dout), jnp.float32),
    )(out_raw, st, bnpack)


def _pad128(v):
    return jnp.zeros((128,), jnp.float32).at[:v.shape[0]].set(v)


def _layer(x, p, meta, gidx, sidx, mpack, tau, alpha, relu):
    din = x.shape[1]
    dout = p['W_fc'].shape[1]
    wproj = jnp.zeros((din, 128), jnp.float32)
    wproj = wproj.at[:, 0].set(p['W_outf'][:, 0]).at[:, 1].set(p['W_inf'][:, 0])
    wprojT = jnp.zeros((8, din), jnp.float32)
    wprojT = wprojT.at[0].set(p['W_outf'][:, 0]).at[1].set(p['W_inf'][:, 0])
    bpack = jnp.stack([
        _pad128(p['b_s2d']),
        _pad128(p['b_d2s']),
        _pad128(p['b_fc']),
        _pad128(jnp.stack([p['b_outf'][0], p['b_inf'][0], tau, alpha])),
        jnp.zeros((128,), jnp.float32),
        jnp.zeros((128,), jnp.float32),
        jnp.zeros((128,), jnp.float32),
        jnp.zeros((128,), jnp.float32),
    ])
    bnpack = jnp.stack([
        _pad128(p['bn_g']), _pad128(p['bn_b']),
        jnp.zeros((128,), jnp.float32), jnp.zeros((128,), jnp.float32),
        jnp.zeros((128,), jnp.float32), jnp.zeros((128,), jnp.float32),
        jnp.zeros((128,), jnp.float32), jnp.zeros((128,), jnp.float32),
    ])

    t1q = _tc_pre_a(x, meta, wproj)
    txw = _tc_pre_b(x, meta, p['W_s2d'], p['W_d2s'], dout)
    z1q = jnp.zeros((ZR, F1Q), jnp.float32)
    zxw = jnp.zeros((ZR, dout), jnp.float32)
    P1q = _sc_spmm(t1q.reshape(NPASS * N, F1Q), gidx, sidx, z1q, F1Q)
    Pxw = _sc_spmm(txw.reshape(NPASS * N, dout), gidx, sidx, zxw, dout)
    gates = _tc_gates(x, meta, mpack, P1q, p['out_emb'], p['in_emb'],
                      wproj, wprojT, bpack)
    out_raw, st = _tc_combine(x, meta, gates, Pxw, p['W_fc'], bpack, dout)
    return _tc_post_c(out_raw, st, bnpack, dout, relu)


def kernel(x, params, edge_indices, edge_types, mask_out, mask_out_b,
           mask_in, mask_in_b):
    npad = E_PAD - E
    padg = jnp.arange(npad, dtype=jnp.int32) % EMB
    pads = N + jnp.arange(npad, dtype=jnp.int32) % (ACC - N)
    gl, sl = [], []
    for i in range(NET):
        row = edge_indices[i, 0]
        col = edge_indices[i, 1]
        gl += [col, row]
        sl += [row, col]
    poff = jnp.arange(NPASS, dtype=jnp.int32)[:, None, None, None, None]
    gidx = jnp.stack([jnp.concatenate([g, padg]) for g in gl])
    gidx = gidx.reshape(NPASS, NSC, NTILE, NCHUNK, B) + poff * N
    sidx = jnp.stack([jnp.concatenate([s_, pads]) for s_ in sl])
    sidx = sidx.reshape(NPASS, NSC, NTILE, NCHUNK, B)
    hidx = sidx + poff * ACC

    mpack = jnp.stack([mask_out[0], mask_out_b[0], mask_in[0], mask_in_b[0],
                       mask_out[1], mask_out_b[1], mask_in[1], mask_in_b[1]],
                      axis=1)

    zrow = jnp.zeros((NPASS * ACC // NTILE,), jnp.float32)
    ones_h = jnp.ones((B,), jnp.float32)
    degP = _sc_hist(hidx, zrow, ones_h).reshape(NSC, NPASS, ACC)
    meta = _tc_degpost(jnp.transpose(degP, (2, 0, 1)))[:N]

    tau = params['tau']
    alpha = params['alpha']
    h = _layer(x, params['conv0'], meta, gidx, sidx, mpack, tau, alpha, True)
    return _layer(h, params['conv1'], meta, gidx, sidx, mpack, tau, alpha,
                  False)


# speedup vs baseline: 1.0535x; 1.0535x over previous
import functools

import jax
import jax.numpy as jnp
from jax import lax
from jax.experimental import pallas as pl
from jax.experimental.pallas import tpu as pltpu
from jax.experimental.pallas import tpu_sc as plsc

N = 10000
E = 160000
NET = 2
DIN = 128
EMB = 256

NSC = 2
NTILE = 16
B = 112
NCHUNK = 46
E_PAD = NSC * NTILE * NCHUNK * B
ACC = 10240
RPT = ACC // NTILE
ZR = 160
NPASS = 2 * NET
F1Q = 136

BN = 400
NBLK = N // BN


def _sc_hist(hidx, zrow, ones_h):
    mesh = plsc.VectorSubcoreMesh(core_axis_name="c", subcore_axis_name="s",
                                  num_cores=NSC, num_subcores=NTILE)
    PT = NPASS * ACC // NTILE

    @functools.partial(
        pl.kernel,
        out_type=jax.ShapeDtypeStruct((NSC * NPASS * ACC,), jnp.float32),
        mesh=mesh,
        scratch_types=[
            pltpu.VMEM_SHARED((NPASS * ACC,), jnp.float32),
            pltpu.VMEM((NCHUNK, B), jnp.int32),
            pltpu.VMEM((B,), jnp.float32),
        ],
    )
    def k(hidx_h, zrow_h, ones_hbm, out, acc, sv, ones_v):
        cid = lax.axis_index("c")
        sid = lax.axis_index("s")
        r0 = sid * RPT
        pltpu.sync_copy(ones_hbm, ones_v)
        pltpu.sync_copy(zrow_h, acc.at[pl.ds(sid * PT, PT)])
        plsc.subcore_barrier()
        for p in range(NPASS):
            pltpu.sync_copy(hidx_h.at[p, cid, sid], sv)

            def body(j, carry):
                pltpu.sync_copy(ones_v, acc.at[sv.at[j]], add=True)
                return carry

            lax.fori_loop(0, NCHUNK, body, 0)
        plsc.subcore_barrier()
        for p in range(NPASS):
            pltpu.sync_copy(
                acc.at[pl.ds(p * ACC + r0, RPT)],
                out.at[pl.ds((cid * NPASS + p) * ACC + r0, RPT)])

    return k(hidx, zrow, ones_h)


def _sc_spmm(tab, gidx, sidx, zrows, F):
    mesh = plsc.VectorSubcoreMesh(core_axis_name="c", subcore_axis_name="s",
                                  num_cores=NSC, num_subcores=NTILE)

    @functools.partial(
        pl.kernel,
        out_type=jax.ShapeDtypeStruct((NSC, NPASS, ACC, F), jnp.float32),
        mesh=mesh,
        scratch_types=[
            pltpu.VMEM_SHARED((ACC, F), jnp.float32),
            pltpu.VMEM((NCHUNK, B), jnp.int32),
            pltpu.VMEM((NCHUNK, B), jnp.int32),
            pltpu.VMEM((B, F), jnp.float32),
            pltpu.VMEM((B, F), jnp.float32),
            pltpu.SemaphoreType.DMA,
            pltpu.SemaphoreType.DMA,
        ],
        compiler_params=pltpu.CompilerParams(use_tc_tiling_on_sc=False),
    )
    def k(tab_h, gidx_h, sidx_h, zrows_h, out, acc, gv, sv, rows0, rows1,
          gsem, ssem):
        cid = lax.axis_index("c")
        sid = lax.axis_index("s")
        r0 = sid * RPT
        rows = (rows0, rows1)
        for p in range(NPASS):
            pltpu.sync_copy(gidx_h.at[p, cid, sid], gv)
            pltpu.sync_copy(sidx_h.at[p, cid, sid], sv)
            for z in range(RPT // ZR):
                pltpu.sync_copy(zrows_h, acc.at[pl.ds(r0 + z * ZR, ZR)])
            plsc.subcore_barrier()

            pltpu.async_copy(tab_h.at[gv.at[0]], rows0, gsem).wait()

            def body(k2, carry):
                for t in range(2):
                    j = 2 * k2 + 1 + t
                    buf = rows[(1 + t) % 2]
                    prev = rows[t % 2]
                    pltpu.async_copy(tab_h.at[gv.at[j]], buf, gsem)
                    pltpu.async_copy(prev, acc.at[sv.at[j - 1]], ssem,
                                     add=True)
                    pltpu.make_async_copy(tab_h.at[gv.at[j]], buf,
                                          gsem).wait()
                    pltpu.make_async_copy(prev, acc.at[sv.at[j - 1]],
                                          ssem).wait()
                return carry

            lax.fori_loop(0, (NCHUNK - 2) // 2, body, 0)
            jl = NCHUNK - 1
            pltpu.async_copy(tab_h.at[gv.at[jl]], rows1, gsem)
            pltpu.async_copy(rows0, acc.at[sv.at[jl - 1]], ssem, add=True)
            pltpu.make_async_copy(tab_h.at[gv.at[jl]], rows1, gsem).wait()
            pltpu.make_async_copy(rows0, acc.at[sv.at[jl - 1]], ssem).wait()
            pltpu.sync_copy(rows1, acc.at[sv.at[jl]], add=True)
            plsc.subcore_barrier()
            pltpu.sync_copy(acc.at[pl.ds(r0, RPT)],
                            out.at[cid, p, pl.ds(r0, RPT)])

    return k(tab, gidx, sidx, zrows)


def _tc_degpost(degPT):
    HB = 1280

    def f(dp_ref, meta_ref):
        dp = dp_ref[...]
        d = dp[:, 0, :] + dp[:, 1, :]
        invd = jnp.where(d > 0.0,
                         1.0 / jnp.sqrt(jnp.maximum(d, 1.0)), 0.0)
        degc = jnp.minimum(d, float(EMB - 1))
        meta_ref[...] = jnp.concatenate([invd, degc], axis=1)

    return pl.pallas_call(
        f,
        grid=(ACC // HB,),
        in_specs=[pl.BlockSpec((HB, NSC, NPASS), lambda j: (j, 0, 0))],
        out_specs=pl.BlockSpec((HB, 8), lambda j: (j, 0)),
        out_shape=jax.ShapeDtypeStruct((ACC, 8), jnp.float32),
    )(degPT)


def _tc_pre_a(x, meta, wproj):
    din = x.shape[1]

    def f(x_ref, m_ref, wp_ref, t1q_ref):
        xb = x_ref[...]
        m = m_ref[...]
        s = 1.0 / (jnp.sum(jnp.abs(xb), axis=1, keepdims=True) + 1e-12)
        x1 = xb * s
        xx = xb * xb
        q2 = jnp.dot(xx, wp_ref[...], preferred_element_type=jnp.float32)
        qo = q2[:, 0:1] * s * s
        qi = q2[:, 1:2] * s * s
        for i in range(NET):
            a = m[:, 2 * i:2 * i + 1]
            b = m[:, 2 * i + 1:2 * i + 2]
            t1q_ref[2 * i] = jnp.concatenate(
                [b * x1, jnp.broadcast_to(b * qo, (BN, F1Q - din))], axis=1)
            t1q_ref[2 * i + 1] = jnp.concatenate(
                [a * x1, jnp.broadcast_to(a * qi, (BN, F1Q - din))], axis=1)

    return pl.pallas_call(
        f,
        grid=(NBLK,),
        in_specs=[
            pl.BlockSpec((BN, din), lambda j: (j, 0)),
            pl.BlockSpec((BN, 8), lambda j: (j, 0)),
            pl.BlockSpec((din, 128), lambda j: (0, 0)),
        ],
        out_specs=pl.BlockSpec((NPASS, BN, F1Q), lambda j: (0, j, 0)),
        out_shape=jax.ShapeDtypeStruct((NPASS, N, F1Q), jnp.float32),
    )(x, meta, wproj)


def _tc_pre_b(x, meta, Ws2d, Wd2s, dout):
    din = x.shape[1]

    def f(x_ref, m_ref, ws_ref, wd_ref, txw_ref):
        xb = x_ref[...]
        m = m_ref[...]
        xws = jnp.dot(xb, ws_ref[...], preferred_element_type=jnp.float32)
        xwd = jnp.dot(xb, wd_ref[...], preferred_element_type=jnp.float32)
        for i in range(NET):
            a = m[:, 2 * i:2 * i + 1]
            b = m[:, 2 * i + 1:2 * i + 2]
            txw_ref[2 * i] = b * xws
            txw_ref[2 * i + 1] = a * xwd

    return pl.pallas_call(
        f,
        grid=(NBLK,),
        in_specs=[
            pl.BlockSpec((BN, din), lambda j: (j, 0)),
            pl.BlockSpec((BN, 8), lambda j: (j, 0)),
            pl.BlockSpec((din, dout), lambda j: (0, 0)),
            pl.BlockSpec((din, dout), lambda j: (0, 0)),
        ],
        out_specs=pl.BlockSpec((NPASS, BN, dout), lambda j: (0, j, 0)),
        out_shape=jax.ShapeDtypeStruct((NPASS, N, dout), jnp.float32),
    )(x, meta, Ws2d, Wd2s)


def _tc_gates(x, meta, mpack, P1q, out_emb, in_emb, wproj, wprojT, bpack):
    din = x.shape[1]

    def f(x_ref, m_ref, mp_ref, p1_ref, oe_ref, ie_ref, wp_ref, wt_ref,
          bp_ref, g_ref):
        xb = x_ref[...]
        m = m_ref[...]
        mp = mp_ref[...]
        bp = bp_ref[...]
        jblk = pl.program_id(0)
        s = 1.0 / (jnp.sum(jnp.abs(xb), axis=1, keepdims=True) + 1e-12)
        x1 = xb * s
        xx = xb * xb
        q2 = jnp.dot(xx, wp_ref[...], preferred_element_type=jnp.float32)
        qo = q2[:, 0:1] * s * s
        qi = q2[:, 1:2] * s * s
        rowi = lax.broadcasted_iota(jnp.int32, (BN, 1), 0) + jblk * BN
        tmd = 1.0 + (rowi == N - 1).astype(jnp.float32)
        embp_o = jnp.dot(oe_ref[...], wp_ref[...],
                         preferred_element_type=jnp.float32)[:, 0:1]
        embp_i = jnp.dot(ie_ref[...], wp_ref[...],
                         preferred_element_type=jnp.float32)[:, 1:2]
        eff_tau = jnp.exp(bp[3, 2]) + 0.1
        emb_iota = lax.broadcasted_iota(jnp.int32, (BN, EMB), 1)
        cols = []
        for i in range(NET):
            a = m[:, 2 * i:2 * i + 1]
            b = m[:, 2 * i + 1:2 * i + 2]
            G1A = p1_ref[0, 2 * i] + p1_ref[1, 2 * i]
            G1B = p1_ref[0, 2 * i + 1] + p1_ref[1, 2 * i + 1]
            eout = (tmd * qo + a * G1A[:, din:din + 1]
                    - 2.0 * jnp.sum(a * G1A[:, :din] * x1 * wt_ref[0:1, :],
                                    axis=1, keepdims=True))
            ein = (tmd * qi + b * G1B[:, din:din + 1]
                   - 2.0 * jnp.sum(b * G1B[:, :din] * x1 * wt_ref[1:2, :],
                                   axis=1, keepdims=True))
            oh_o = (m[:, 4 + 2 * i:5 + 2 * i].astype(jnp.int32)
                    == emb_iota).astype(jnp.float32)
            oh_i = (m[:, 5 + 2 * i:6 + 2 * i].astype(jnp.int32)
                    == emb_iota).astype(jnp.float32)
            enc_o = jnp.dot(oh_o, embp_o, preferred_element_type=jnp.float32)
            enc_i = jnp.dot(oh_i, embp_i, preferred_element_type=jnp.float32)
            co = -eout + enc_o + bp[3, 0]
            ci = -ein + enc_i + bp[3, 1]
            u = co / eff_tau
            v = ci / eff_tau
            mx = jnp.maximum(u, v)
            eu = jnp.exp(u - mx)
            ev = jnp.exp(v - mx)
            z = eu + ev
            cols.append((eu / z) * mp[:, 4 * i:4 * i + 1]
                        + mp[:, 4 * i + 1:4 * i + 2])
            cols.append((ev / z) * mp[:, 4 * i + 2:4 * i + 3]
                        + mp[:, 4 * i + 3:4 * i + 4])
        cols.append(jnp.zeros((BN, 8 - 2 * NET), jnp.float32))
        g_ref[...] = jnp.concatenate(cols, axis=1)

    return pl.pallas_call(
        f,
        grid=(NBLK,),
        in_specs=[
            pl.BlockSpec((BN, din), lambda j: (j, 0)),
            pl.BlockSpec((BN, 8), lambda j: (j, 0)),
            pl.BlockSpec((BN, 8), lambda j: (j, 0)),
            pl.BlockSpec((NSC, NPASS, BN, F1Q), lambda j: (0, 0, j, 0)),
            pl.BlockSpec((EMB, din), lambda j: (0, 0)),
            pl.BlockSpec((EMB, din), lambda j: (0, 0)),
            pl.BlockSpec((din, 128), lambda j: (0, 0)),
            pl.BlockSpec((8, din), lambda j: (0, 0)),
            pl.BlockSpec((8, 128), lambda j: (0, 0)),
        ],
        out_specs=pl.BlockSpec((BN, 8), lambda j: (j, 0)),
        out_shape=jax.ShapeDtypeStruct((N, 8), jnp.float32),
    )(x, meta, mpack, P1q, out_emb, in_emb, wproj, wprojT, bpack)


def _tc_combine(x, meta, gates, Pxw, Wfc, bpack, dout):
    din = x.shape[1]

    def f(x_ref, m_ref, g_ref, pw_ref, wfc_ref, bp_ref, out_ref, st_ref):
        xb = x_ref[...]
        m = m_ref[...]
        g = g_ref[...]
        bp = bp_ref[...]
        alpha = bp[3, 3]
        acc = alpha * (jnp.dot(xb, wfc_ref[...],
                               preferred_element_type=jnp.float32)
                       + bp[2:3, :dout])
        for i in range(NET):
            a = m[:, 2 * i:2 * i + 1]
            b = m[:, 2 * i + 1:2 * i + 2]
            GWA = pw_ref[0, 2 * i] + pw_ref[1, 2 * i]
            GWB = pw_ref[0, 2 * i + 1] + pw_ref[1, 2 * i + 1]
            out_nei = a * GWA + bp[0:1, :dout]
            in_nei = b * GWB + bp[1:2, :dout]
            acc = acc + (1.0 / NET) * (g[:, 2 * i:2 * i + 1] * out_nei
                                       + g[:, 2 * i + 1:2 * i + 2] * in_nei)
        out_ref[...] = acc
        st_ref[...] = jnp.zeros((1, 8, dout), jnp.float32)
        st_ref[0, 0] = jnp.sum(acc, axis=0)
        st_ref[0, 1] = jnp.sum(acc * acc, axis=0)

    return pl.pallas_call(
        f,
        grid=(NBLK,),
        in_specs=[
            pl.BlockSpec((BN, din), lambda j: (j, 0)),
            pl.BlockSpec((BN, 8), lambda j: (j, 0)),
            pl.BlockSpec((BN, 8), lambda j: (j, 0)),
            pl.BlockSpec((NSC, NPASS, BN, dout), lambda j: (0, 0, j, 0)),
            pl.BlockSpec((din, dout), lambda j: (0, 0)),
            pl.BlockSpec((8, 128), lambda j: (0, 0)),
        ],
        out_specs=[
            pl.BlockSpec((BN, dout), lambda j: (j, 0)),
            pl.BlockSpec((1, 8, dout), lambda j: (j, 0, 0)),
        ],
        out_shape=[
            jax.ShapeDtypeStruct((N, dout), jnp.float32),
            jax.ShapeDtypeStruct((NBLK, 8, dout), jnp.float32),
        ],
    )(x, meta, gates, Pxw, Wfc, bpack)


def _tc_post_c(out_raw, st, bnpack, dout, relu):
    def f(o_ref, st_ref, bn_ref, y_ref):
        stats = st_ref[...]
        mean = jnp.sum(stats[:, 0, :], axis=0, keepdims=True) / N
        msq = jnp.sum(stats[:, 1, :], axis=0, keepdims=True) / N
        var = msq - mean * mean
        inv = 1.0 / jnp.sqrt(var + 1e-5)
        bn = bn_ref[...]
        y = (o_ref[...] - mean) * inv * bn[0:1, :dout] + bn[1:2, :dout]
        if relu:
            y = jnp.maximum(y, 0.0)
        y_ref[...] = y

    return pl.pallas_call(
        f,
        grid=(NBLK,),
        in_specs=[
            pl.BlockSpec((BN, dout), lambda j: (j, 0)),
            pl.BlockSpec((NBLK, 8, dout), lambda j: (0, 0, 0)),
            pl.BlockSpec((8, 128), lambda j: (0, 0)),
        ],
        out_specs=pl.BlockSpec((BN, dout), lambda j: (j, 0)),
        out_shape=jax.ShapeDtypeStruct((N, dout), jnp.float32),
    )(out_raw, st, bnpack)


def _pad128(v):
    return jnp.zeros((128,), jnp.float32).at[:v.shape[0]].set(v)


def _layer(x, p, meta, gidx, sidx, mpack, tau, alpha, relu):
    din = x.shape[1]
    dout = p['W_fc'].shape[1]
    wproj = jnp.zeros((din, 128), jnp.float32)
    wproj = wproj.at[:, 0].set(p['W_outf'][:, 0]).at[:, 1].set(p['W_inf'][:, 0])
    wprojT = jnp.zeros((8, din), jnp.float32)
    wprojT = wprojT.at[0].set(p['W_outf'][:, 0]).at[1].set(p['W_inf'][:, 0])
    bpack = jnp.stack([
        _pad128(p['b_s2d']),
        _pad128(p['b_d2s']),
        _pad128(p['b_fc']),
        _pad128(jnp.stack([p['b_outf'][0], p['b_inf'][0], tau, alpha])),
        jnp.zeros((128,), jnp.float32),
        jnp.zeros((128,), jnp.float32),
        jnp.zeros((128,), jnp.float32),
        jnp.zeros((128,), jnp.float32),
    ])
    bnpack = jnp.stack([
        _pad128(p['bn_g']), _pad128(p['bn_b']),
        jnp.zeros((128,), jnp.float32), jnp.zeros((128,), jnp.float32),
        jnp.zeros((128,), jnp.float32), jnp.zeros((128,), jnp.float32),
        jnp.zeros((128,), jnp.float32), jnp.zeros((128,), jnp.float32),
    ])

    t1q = _tc_pre_a(x, meta, wproj)
    txw = _tc_pre_b(x, meta, p['W_s2d'], p['W_d2s'], dout)
    z1q = jnp.zeros((ZR, F1Q), jnp.float32)
    zxw = jnp.zeros((ZR, dout), jnp.float32)
    P1q = _sc_spmm(t1q.reshape(NPASS * N, F1Q), gidx, sidx, z1q, F1Q)
    Pxw = _sc_spmm(txw.reshape(NPASS * N, dout), gidx, sidx, zxw, dout)
    gates = _tc_gates(x, meta, mpack, P1q, p['out_emb'], p['in_emb'],
                      wproj, wprojT, bpack)
    out_raw, st = _tc_combine(x, meta, gates, Pxw, p['W_fc'], bpack, dout)
    return _tc_post_c(out_raw, st, bnpack, dout, relu)


def kernel(x, params, edge_indices, edge_types, mask_out, mask_out_b,
           mask_in, mask_in_b):
    npad = E_PAD - E
    padg = jnp.arange(npad, dtype=jnp.int32) % EMB
    pads = N + jnp.arange(npad, dtype=jnp.int32) % (ACC - N)
    gl, sl = [], []
    for i in range(NET):
        row = edge_indices[i, 0]
        col = edge_indices[i, 1]
        gl += [col, row]
        sl += [row, col]
    poff = jnp.arange(NPASS, dtype=jnp.int32)[:, None, None, None, None]
    gidx = jnp.stack([jnp.concatenate([g, padg]) for g in gl])
    gidx = gidx.reshape(NPASS, NSC, NTILE, NCHUNK, B) + poff * N
    sidx = jnp.stack([jnp.concatenate([s_, pads]) for s_ in sl])
    sidx = sidx.reshape(NPASS, NSC, NTILE, NCHUNK, B)
    hidx = sidx + poff * ACC

    mpack = jnp.stack([mask_out[0], mask_out_b[0], mask_in[0], mask_in_b[0],
                       mask_out[1], mask_out_b[1], mask_in[1], mask_in_b[1]],
                      axis=1)

    zrow = jnp.zeros((NPASS * ACC // NTILE,), jnp.float32)
    ones_h = jnp.ones((B,), jnp.float32)
    degP = _sc_hist(hidx, zrow, ones_h).reshape(NSC, NPASS, ACC)
    meta = _tc_degpost(jnp.transpose(degP, (2, 0, 1)))[:N]

    tau = params['tau']
    alpha = params['alpha']
    h = _layer(x, params['conv0'], meta, gidx, sidx, mpack, tau, alpha, True)
    return _layer(h, params['conv1'], meta, gidx, sidx, mpack, tau, alpha,
                  False)


# speedup vs baseline: 1.0877x; 1.0325x over previous
import functools

import jax
import jax.numpy as jnp
from jax import lax
from jax.experimental import pallas as pl
from jax.experimental.pallas import tpu as pltpu
from jax.experimental.pallas import tpu_sc as plsc

N = 10000
E = 160000
NET = 2
DIN = 128
EMB = 256

NSC = 2
NTILE = 16
B = 128
NCHUNK = 40
E_PAD = NSC * NTILE * NCHUNK * B
ACC = 10240
RPT = ACC // NTILE
SACC = 10112
SRPT = SACC // NTILE
SZR = 158
NPASS = 2 * NET
F1Q = 136

BN = 400
NBLK = N // BN


def _sc_hist(hidx, zrow, ones_h):
    mesh = plsc.VectorSubcoreMesh(core_axis_name="c", subcore_axis_name="s",
                                  num_cores=NSC, num_subcores=NTILE)
    PT = NPASS * ACC // NTILE

    @functools.partial(
        pl.kernel,
        out_type=jax.ShapeDtypeStruct((NSC * NPASS * ACC,), jnp.float32),
        mesh=mesh,
        scratch_types=[
            pltpu.VMEM_SHARED((NPASS * ACC,), jnp.float32),
            pltpu.VMEM((NCHUNK, B), jnp.int32),
            pltpu.VMEM((B,), jnp.float32),
        ],
    )
    def k(hidx_h, zrow_h, ones_hbm, out, acc, sv, ones_v):
        cid = lax.axis_index("c")
        sid = lax.axis_index("s")
        r0 = sid * RPT
        pltpu.sync_copy(ones_hbm, ones_v)
        pltpu.sync_copy(zrow_h, acc.at[pl.ds(sid * PT, PT)])
        plsc.subcore_barrier()
        for p in range(NPASS):
            pltpu.sync_copy(hidx_h.at[p, cid, sid], sv)

            def body(j, carry):
                pltpu.sync_copy(ones_v, acc.at[sv.at[j]], add=True)
                return carry

            lax.fori_loop(0, NCHUNK, body, 0)
        plsc.subcore_barrier()
        for p in range(NPASS):
            pltpu.sync_copy(
                acc.at[pl.ds(p * ACC + r0, RPT)],
                out.at[pl.ds((cid * NPASS + p) * ACC + r0, RPT)])

    return k(hidx, zrow, ones_h)


def _sc_spmm(tab, gidx, sidx, zrows, F):
    mesh = plsc.VectorSubcoreMesh(core_axis_name="c", subcore_axis_name="s",
                                  num_cores=NSC, num_subcores=NTILE)

    @functools.partial(
        pl.kernel,
        out_type=jax.ShapeDtypeStruct((NSC, NPASS, SACC, F), jnp.float32),
        mesh=mesh,
        scratch_types=[
            pltpu.VMEM_SHARED((SACC, F), jnp.float32),
            pltpu.VMEM((NCHUNK, B), jnp.int32),
            pltpu.VMEM((NCHUNK, B), jnp.int32),
            pltpu.VMEM((B, F), jnp.float32),
            pltpu.VMEM((B, F), jnp.float32),
            pltpu.SemaphoreType.DMA,
            pltpu.SemaphoreType.DMA,
        ],
        compiler_params=pltpu.CompilerParams(use_tc_tiling_on_sc=False),
    )
    def k(tab_h, gidx_h, sidx_h, zrows_h, out, acc, gv, sv, rows0, rows1,
          gsem, ssem):
        cid = lax.axis_index("c")
        sid = lax.axis_index("s")
        r0 = sid * SRPT
        rows = (rows0, rows1)
        for p in range(NPASS):
            pltpu.sync_copy(gidx_h.at[p, cid, sid], gv)
            pltpu.sync_copy(sidx_h.at[p, cid, sid], sv)
            for z in range(SRPT // SZR):
                pltpu.sync_copy(zrows_h, acc.at[pl.ds(r0 + z * SZR, SZR)])
            plsc.subcore_barrier()

            pltpu.async_copy(tab_h.at[gv.at[0]], rows0, gsem).wait()

            def body(k2, carry):
                for t in range(2):
                    j = 2 * k2 + 1 + t
                    buf = rows[(1 + t) % 2]
                    prev = rows[t % 2]
                    pltpu.async_copy(tab_h.at[gv.at[j]], buf, gsem)
                    pltpu.async_copy(prev, acc.at[sv.at[j - 1]], ssem,
                                     add=True)
                    pltpu.make_async_copy(tab_h.at[gv.at[j]], buf,
                                          gsem).wait()
                    pltpu.make_async_copy(prev, acc.at[sv.at[j - 1]],
                                          ssem).wait()
                return carry

            lax.fori_loop(0, (NCHUNK - 2) // 2, body, 0)
            jl = NCHUNK - 1
            pltpu.async_copy(tab_h.at[gv.at[jl]], rows1, gsem)
            pltpu.async_copy(rows0, acc.at[sv.at[jl - 1]], ssem, add=True)
            pltpu.make_async_copy(tab_h.at[gv.at[jl]], rows1, gsem).wait()
            pltpu.make_async_copy(rows0, acc.at[sv.at[jl - 1]], ssem).wait()
            pltpu.sync_copy(rows1, acc.at[sv.at[jl]], add=True)
            plsc.subcore_barrier()
            pltpu.sync_copy(acc.at[pl.ds(r0, SRPT)],
                            out.at[cid, p, pl.ds(r0, SRPT)])

    return k(tab, gidx, sidx, zrows)


def _tc_degpost(degPT):
    HB = 1280

    def f(dp_ref, meta_ref):
        dp = dp_ref[...]
        d = dp[:, 0, :] + dp[:, 1, :]
        invd = jnp.where(d > 0.0,
                         1.0 / jnp.sqrt(jnp.maximum(d, 1.0)), 0.0)
        degc = jnp.minimum(d, float(EMB - 1))
        meta_ref[...] = jnp.concatenate([invd, degc], axis=1)

    return pl.pallas_call(
        f,
        grid=(ACC // HB,),
        in_specs=[pl.BlockSpec((HB, NSC, NPASS), lambda j: (j, 0, 0))],
        out_specs=pl.BlockSpec((HB, 8), lambda j: (j, 0)),
        out_shape=jax.ShapeDtypeStruct((ACC, 8), jnp.float32),
    )(degPT)


def _tc_pre_a(x, meta, wproj):
    din = x.shape[1]

    def f(x_ref, m_ref, wp_ref, t1q_ref):
        xb = x_ref[...]
        m = m_ref[...]
        s = 1.0 / (jnp.sum(jnp.abs(xb), axis=1, keepdims=True) + 1e-12)
        x1 = xb * s
        xx = xb * xb
        q2 = jnp.dot(xx, wp_ref[...], preferred_element_type=jnp.float32)
        qo = q2[:, 0:1] * s * s
        qi = q2[:, 1:2] * s * s
        for i in range(NET):
            a = m[:, 2 * i:2 * i + 1]
            b = m[:, 2 * i + 1:2 * i + 2]
            t1q_ref[2 * i] = jnp.concatenate(
                [b * x1, jnp.broadcast_to(b * qo, (BN, F1Q - din))], axis=1)
            t1q_ref[2 * i + 1] = jnp.concatenate(
                [a * x1, jnp.broadcast_to(a * qi, (BN, F1Q - din))], axis=1)

    return pl.pallas_call(
        f,
        grid=(NBLK,),
        in_specs=[
            pl.BlockSpec((BN, din), lambda j: (j, 0)),
            pl.BlockSpec((BN, 8), lambda j: (j, 0)),
            pl.BlockSpec((din, 128), lambda j: (0, 0)),
        ],
        out_specs=pl.BlockSpec((NPASS, BN, F1Q), lambda j: (0, j, 0)),
        out_shape=jax.ShapeDtypeStruct((NPASS, N, F1Q), jnp.float32),
    )(x, meta, wproj)


def _tc_pre_b(x, meta, Ws2d, Wd2s, dout):
    din = x.shape[1]

    def f(x_ref, m_ref, ws_ref, wd_ref, txw_ref):
        xb = x_ref[...]
        m = m_ref[...]
        xws = jnp.dot(xb, ws_ref[...], preferred_element_type=jnp.float32)
        xwd = jnp.dot(xb, wd_ref[...], preferred_element_type=jnp.float32)
        for i in range(NET):
            a = m[:, 2 * i:2 * i + 1]
            b = m[:, 2 * i + 1:2 * i + 2]
            txw_ref[2 * i] = b * xws
            txw_ref[2 * i + 1] = a * xwd

    return pl.pallas_call(
        f,
        grid=(NBLK,),
        in_specs=[
            pl.BlockSpec((BN, din), lambda j: (j, 0)),
            pl.BlockSpec((BN, 8), lambda j: (j, 0)),
            pl.BlockSpec((din, dout), lambda j: (0, 0)),
            pl.BlockSpec((din, dout), lambda j: (0, 0)),
        ],
        out_specs=pl.BlockSpec((NPASS, BN, dout), lambda j: (0, j, 0)),
        out_shape=jax.ShapeDtypeStruct((NPASS, N, dout), jnp.float32),
    )(x, meta, Ws2d, Wd2s)


def _tc_gates(x, meta, mpack, P1q, out_emb, in_emb, wproj, wprojT, bpack):
    din = x.shape[1]

    def f(x_ref, m_ref, mp_ref, p1_ref, oe_ref, ie_ref, wp_ref, wt_ref,
          bp_ref, g_ref):
        xb = x_ref[...]
        m = m_ref[...]
        mp = mp_ref[...]
        bp = bp_ref[...]
        jblk = pl.program_id(0)
        s = 1.0 / (jnp.sum(jnp.abs(xb), axis=1, keepdims=True) + 1e-12)
        x1 = xb * s
        xx = xb * xb
        q2 = jnp.dot(xx, wp_ref[...], preferred_element_type=jnp.float32)
        qo = q2[:, 0:1] * s * s
        qi = q2[:, 1:2] * s * s
        rowi = lax.broadcasted_iota(jnp.int32, (BN, 1), 0) + jblk * BN
        tmd = 1.0 + (rowi == N - 1).astype(jnp.float32)
        embp_o = jnp.dot(oe_ref[...], wp_ref[...],
                         preferred_element_type=jnp.float32)[:, 0:1]
        embp_i = jnp.dot(ie_ref[...], wp_ref[...],
                         preferred_element_type=jnp.float32)[:, 1:2]
        eff_tau = jnp.exp(bp[3, 2]) + 0.1
        emb_iota = lax.broadcasted_iota(jnp.int32, (BN, EMB), 1)
        cols = []
        for i in range(NET):
            a = m[:, 2 * i:2 * i + 1]
            b = m[:, 2 * i + 1:2 * i + 2]
            G1A = p1_ref[0, 2 * i] + p1_ref[1, 2 * i]
            G1B = p1_ref[0, 2 * i + 1] + p1_ref[1, 2 * i + 1]
            eout = (tmd * qo + a * G1A[:, din:din + 1]
                    - 2.0 * jnp.sum(a * G1A[:, :din] * x1 * wt_ref[0:1, :],
                                    axis=1, keepdims=True))
            ein = (tmd * qi + b * G1B[:, din:din + 1]
                   - 2.0 * jnp.sum(b * G1B[:, :din] * x1 * wt_ref[1:2, :],
                                   axis=1, keepdims=True))
            oh_o = (m[:, 4 + 2 * i:5 + 2 * i].astype(jnp.int32)
                    == emb_iota).astype(jnp.float32)
            oh_i = (m[:, 5 + 2 * i:6 + 2 * i].astype(jnp.int32)
                    == emb_iota).astype(jnp.float32)
            enc_o = jnp.dot(oh_o, embp_o, preferred_element_type=jnp.float32)
            enc_i = jnp.dot(oh_i, embp_i, preferred_element_type=jnp.float32)
            co = -eout + enc_o + bp[3, 0]
            ci = -ein + enc_i + bp[3, 1]
            u = co / eff_tau
            v = ci / eff_tau
            mx = jnp.maximum(u, v)
            eu = jnp.exp(u - mx)
            ev = jnp.exp(v - mx)
            z = eu + ev
            cols.append((eu / z) * mp[:, 4 * i:4 * i + 1]
                        + mp[:, 4 * i + 1:4 * i + 2])
            cols.append((ev / z) * mp[:, 4 * i + 2:4 * i + 3]
                        + mp[:, 4 * i + 3:4 * i + 4])
        cols.append(jnp.zeros((BN, 8 - 2 * NET), jnp.float32))
        g_ref[...] = jnp.concatenate(cols, axis=1)

    return pl.pallas_call(
        f,
        grid=(NBLK,),
        in_specs=[
            pl.BlockSpec((BN, din), lambda j: (j, 0)),
            pl.BlockSpec((BN, 8), lambda j: (j, 0)),
            pl.BlockSpec((BN, 8), lambda j: (j, 0)),
            pl.BlockSpec((NSC, NPASS, BN, F1Q), lambda j: (0, 0, j, 0)),
            pl.BlockSpec((EMB, din), lambda j: (0, 0)),
            pl.BlockSpec((EMB, din), lambda j: (0, 0)),
            pl.BlockSpec((din, 128), lambda j: (0, 0)),
            pl.BlockSpec((8, din), lambda j: (0, 0)),
            pl.BlockSpec((8, 128), lambda j: (0, 0)),
        ],
        out_specs=pl.BlockSpec((BN, 8), lambda j: (j, 0)),
        out_shape=jax.ShapeDtypeStruct((N, 8), jnp.float32),
    )(x, meta, mpack, P1q, out_emb, in_emb, wproj, wprojT, bpack)


def _tc_combine(x, meta, gates, Pxw, Wfc, bpack, dout):
    din = x.shape[1]

    def f(x_ref, m_ref, g_ref, pw_ref, wfc_ref, bp_ref, out_ref, st_ref):
        xb = x_ref[...]
        m = m_ref[...]
        g = g_ref[...]
        bp = bp_ref[...]
        alpha = bp[3, 3]
        acc = alpha * (jnp.dot(xb, wfc_ref[...],
                               preferred_element_type=jnp.float32)
                       + bp[2:3, :dout])
        for i in range(NET):
            a = m[:, 2 * i:2 * i + 1]
            b = m[:, 2 * i + 1:2 * i + 2]
            GWA = pw_ref[0, 2 * i] + pw_ref[1, 2 * i]
            GWB = pw_ref[0, 2 * i + 1] + pw_ref[1, 2 * i + 1]
            out_nei = a * GWA + bp[0:1, :dout]
            in_nei = b * GWB + bp[1:2, :dout]
            acc = acc + (1.0 / NET) * (g[:, 2 * i:2 * i + 1] * out_nei
                                       + g[:, 2 * i + 1:2 * i + 2] * in_nei)
        out_ref[...] = acc
        st_ref[...] = jnp.zeros((1, 8, dout), jnp.float32)
        st_ref[0, 0] = jnp.sum(acc, axis=0)
        st_ref[0, 1] = jnp.sum(acc * acc, axis=0)

    return pl.pallas_call(
        f,
        grid=(NBLK,),
        in_specs=[
            pl.BlockSpec((BN, din), lambda j: (j, 0)),
            pl.BlockSpec((BN, 8), lambda j: (j, 0)),
            pl.BlockSpec((BN, 8), lambda j: (j, 0)),
            pl.BlockSpec((NSC, NPASS, BN, dout), lambda j: (0, 0, j, 0)),
            pl.BlockSpec((din, dout), lambda j: (0, 0)),
            pl.BlockSpec((8, 128), lambda j: (0, 0)),
        ],
        out_specs=[
            pl.BlockSpec((BN, dout), lambda j: (j, 0)),
            pl.BlockSpec((1, 8, dout), lambda j: (j, 0, 0)),
        ],
        out_shape=[
            jax.ShapeDtypeStruct((N, dout), jnp.float32),
            jax.ShapeDtypeStruct((NBLK, 8, dout), jnp.float32),
        ],
    )(x, meta, gates, Pxw, Wfc, bpack)


def _tc_post_c(out_raw, st, bnpack, dout, relu):
    def f(o_ref, st_ref, bn_ref, y_ref):
        stats = st_ref[...]
        mean = jnp.sum(stats[:, 0, :], axis=0, keepdims=True) / N
        msq = jnp.sum(stats[:, 1, :], axis=0, keepdims=True) / N
        var = msq - mean * mean
        inv = 1.0 / jnp.sqrt(var + 1e-5)
        bn = bn_ref[...]
        y = (o_ref[...] - mean) * inv * bn[0:1, :dout] + bn[1:2, :dout]
        if relu:
            y = jnp.maximum(y, 0.0)
        y_ref[...] = y

    return pl.pallas_call(
        f,
        grid=(NBLK,),
        in_specs=[
            pl.BlockSpec((BN, dout), lambda j: (j, 0)),
            pl.BlockSpec((NBLK, 8, dout), lambda j: (0, 0, 0)),
            pl.BlockSpec((8, 128), lambda j: (0, 0)),
        ],
        out_specs=pl.BlockSpec((BN, dout), lambda j: (j, 0)),
        out_shape=jax.ShapeDtypeStruct((N, dout), jnp.float32),
    )(out_raw, st, bnpack)


def _pad128(v):
    return jnp.zeros((128,), jnp.float32).at[:v.shape[0]].set(v)


def _layer(x, p, meta, gidx, sidx, mpack, tau, alpha, relu):
    din = x.shape[1]
    dout = p['W_fc'].shape[1]
    wproj = jnp.zeros((din, 128), jnp.float32)
    wproj = wproj.at[:, 0].set(p['W_outf'][:, 0]).at[:, 1].set(p['W_inf'][:, 0])
    wprojT = jnp.zeros((8, din), jnp.float32)
    wprojT = wprojT.at[0].set(p['W_outf'][:, 0]).at[1].set(p['W_inf'][:, 0])
    bpack = jnp.stack([
        _pad128(p['b_s2d']),
        _pad128(p['b_d2s']),
        _pad128(p['b_fc']),
        _pad128(jnp.stack([p['b_outf'][0], p['b_inf'][0], tau, alpha])),
        jnp.zeros((128,), jnp.float32),
        jnp.zeros((128,), jnp.float32),
        jnp.zeros((128,), jnp.float32),
        jnp.zeros((128,), jnp.float32),
    ])
    bnpack = jnp.stack([
        _pad128(p['bn_g']), _pad128(p['bn_b']),
        jnp.zeros((128,), jnp.float32), jnp.zeros((128,), jnp.float32),
        jnp.zeros((128,), jnp.float32), jnp.zeros((128,), jnp.float32),
        jnp.zeros((128,), jnp.float32), jnp.zeros((128,), jnp.float32),
    ])

    t1q = _tc_pre_a(x, meta, wproj)
    txw = _tc_pre_b(x, meta, p['W_s2d'], p['W_d2s'], dout)
    z1q = jnp.zeros((SZR, F1Q), jnp.float32)
    zxw = jnp.zeros((SZR, dout), jnp.float32)
    P1q = _sc_spmm(t1q.reshape(NPASS * N, F1Q), gidx, sidx, z1q, F1Q)
    Pxw = _sc_spmm(txw.reshape(NPASS * N, dout), gidx, sidx, zxw, dout)
    gates = _tc_gates(x, meta, mpack, P1q, p['out_emb'], p['in_emb'],
                      wproj, wprojT, bpack)
    out_raw, st = _tc_combine(x, meta, gates, Pxw, p['W_fc'], bpack, dout)
    return _tc_post_c(out_raw, st, bnpack, dout, relu)


def kernel(x, params, edge_indices, edge_types, mask_out, mask_out_b,
           mask_in, mask_in_b):
    npad = E_PAD - E
    padg = jnp.arange(npad, dtype=jnp.int32) % EMB
    pads = N + jnp.arange(npad, dtype=jnp.int32) % (SACC - N)
    gl, sl = [], []
    for i in range(NET):
        row = edge_indices[i, 0]
        col = edge_indices[i, 1]
        gl += [col, row]
        sl += [row, col]
    poff = jnp.arange(NPASS, dtype=jnp.int32)[:, None, None, None, None]
    gidx = jnp.stack([jnp.concatenate([g, padg]) for g in gl])
    gidx = gidx.reshape(NPASS, NSC, NTILE, NCHUNK, B) + poff * N
    sidx = jnp.stack([jnp.concatenate([s_, pads]) for s_ in sl])
    sidx = sidx.reshape(NPASS, NSC, NTILE, NCHUNK, B)
    hidx = sidx + poff * ACC

    mpack = jnp.stack([mask_out[0], mask_out_b[0], mask_in[0], mask_in_b[0],
                       mask_out[1], mask_out_b[1], mask_in[1], mask_in_b[1]],
                      axis=1)

    zrow = jnp.zeros((NPASS * ACC // NTILE,), jnp.float32)
    ones_h = jnp.ones((B,), jnp.float32)
    degP = _sc_hist(hidx, zrow, ones_h).reshape(NSC, NPASS, ACC)
    meta = _tc_degpost(jnp.transpose(degP, (2, 0, 1)))[:N]

    tau = params['tau']
    alpha = params['alpha']
    h = _layer(x, params['conv0'], meta, gidx, sidx, mpack, tau, alpha, True)
    return _layer(h, params['conv1'], meta, gidx, sidx, mpack, tau, alpha,
                  False)
